# Initial kernel scaffold; baseline (speedup 1.0000x reference)
import jax
import jax.numpy as jnp
from jax.experimental import pallas as pl


def kernel(x, row_splits, Wp, bp, Ws, bs, Wo, bo):
    raise NotImplementedError("write your pallas kernel here")


# speedup vs baseline: 16.4790x; 16.4790x over previous
import functools

import jax
import jax.numpy as jnp
from jax.experimental import pallas as pl

_K = 40


def _body(x_ref, Wp_ref, bp_ref, Ws_ref, bs_ref, Wo_ref, bo_ref, out_ref):
    S = x_ref.shape[0]
    P = Wp_ref.shape[1]
    ND = Ws_ref.shape[1]

    xb = x_ref[...]
    coords = jnp.dot(xb, Ws_ref[...],
                     preferred_element_type=jnp.float32) + bs_ref[...]
    feat = jnp.dot(xb, Wp_ref[...],
                   preferred_element_type=jnp.float32) + bp_ref[...]

    d2 = jnp.zeros((S, S), dtype=jnp.float32)
    for d in range(ND):
        col = coords[:, d:d + 1]
        diff = col - col.reshape(1, S)
        d2 = d2 + diff * diff

    colids = jax.lax.broadcasted_iota(jnp.int32, (S, S), 1)
    BIG = jnp.float32(3.0e38)
    NEG = jnp.float32(-3.0e38)

    def step(t, carry):
        d2c, fmax, fsum = carry
        m = jnp.min(d2c, axis=1, keepdims=True)
        amin = jnp.min(jnp.where(d2c == m, colids, S), axis=1, keepdims=True)
        onehot = (colids == amin).astype(jnp.float32)
        fnb = jnp.dot(onehot, feat, preferred_element_type=jnp.float32)
        w = jnp.exp(-jnp.abs(10.0 * m))
        wf = w * fnb
        keep = t >= 1
        fmax = jnp.where(keep, jnp.maximum(fmax, wf), fmax)
        fsum = jnp.where(keep, fsum + wf, fsum)
        d2c = jnp.where(colids == amin, BIG, d2c)
        return d2c, fmax, fsum

    init = (d2, jnp.full((S, P), NEG, jnp.float32), jnp.zeros((S, P), jnp.float32))
    _, fmax, fsum = jax.lax.fori_loop(0, _K + 1, step, init)
    fmean = fsum * jnp.float32(1.0 / _K)

    Wo = Wo_ref[...]
    acc = jnp.dot(fmax, Wo[:P, :], preferred_element_type=jnp.float32)
    acc = acc + jnp.dot(fmean, Wo[P:2 * P, :], preferred_element_type=jnp.float32)
    acc = acc + jnp.dot(xb, Wo[2 * P:, :], preferred_element_type=jnp.float32)
    out_ref[...] = jnp.tanh(acc + bo_ref[...])


def kernel(x, row_splits, Wp, bp, Ws, bs, Wo, bo):
    N, D = x.shape
    nseg = int(row_splits.shape[0]) - 1
    S = N // nseg
    P = Wp.shape[1]
    ND = Ws.shape[1]
    F = Wo.shape[1]

    grid = (nseg,)
    out = pl.pallas_call(
        _body,
        grid=grid,
        in_specs=[
            pl.BlockSpec((S, D), lambda b: (b, 0)),
            pl.BlockSpec((D, P), lambda b: (0, 0)),
            pl.BlockSpec((1, P), lambda b: (0, 0)),
            pl.BlockSpec((D, ND), lambda b: (0, 0)),
            pl.BlockSpec((1, ND), lambda b: (0, 0)),
            pl.BlockSpec((2 * P + D, F), lambda b: (0, 0)),
            pl.BlockSpec((1, F), lambda b: (0, 0)),
        ],
        out_specs=pl.BlockSpec((S, F), lambda b: (b, 0)),
        out_shape=jax.ShapeDtypeStruct((N, F), jnp.float32),
    )(x, Wp, bp.reshape(1, P), Ws, bs.reshape(1, ND), Wo, bo.reshape(1, F))
    return out


# speedup vs baseline: 18.6558x; 1.1321x over previous
import functools

import jax
import jax.numpy as jnp
from jax.experimental import pallas as pl

_K = 40


def _body(x_ref, Wp_ref, bp_ref, Ws_ref, bs_ref, Wo_ref, bo_ref, out_ref):
    S = x_ref.shape[0]
    P = Wp_ref.shape[1]
    ND = Ws_ref.shape[1]

    xb = x_ref[...]
    coords = jnp.dot(xb, Ws_ref[...],
                     preferred_element_type=jnp.float32) + bs_ref[...]
    feat = jnp.dot(xb, Wp_ref[...],
                   preferred_element_type=jnp.float32) + bp_ref[...]

    d2 = jnp.zeros((S, S), dtype=jnp.float32)
    for d in range(ND):
        col = coords[:, d:d + 1]
        diff = col - col.reshape(1, S)
        d2 = d2 + diff * diff

    colids = jax.lax.broadcasted_iota(jnp.int32, (S, S), 1)
    NEG = jnp.float32(-3.0e38)
    IMAX = jnp.int32(0x7FFFFFFF)

    packed = jnp.bitwise_or(
        jnp.bitwise_and(jax.lax.bitcast_convert_type(d2, jnp.int32), ~(S - 1)),
        colids)
    feat16 = feat.astype(jnp.bfloat16)

    def step(t, carry):
        pk, fmax, fsum = carry
        m = jnp.min(pk, axis=1, keepdims=True)
        onehot = pk == m
        d2t = jax.lax.bitcast_convert_type(
            jnp.bitwise_and(m, ~(S - 1)), jnp.float32)
        w = jnp.exp(-10.0 * d2t)
        fnb = jnp.dot(onehot.astype(jnp.bfloat16), feat16,
                      preferred_element_type=jnp.float32)
        wf = w * fnb
        keep = t >= 1
        fmax = jnp.where(keep, jnp.maximum(fmax, wf), fmax)
        fsum = jnp.where(keep, fsum + wf, fsum)
        pk = jnp.where(onehot, IMAX, pk)
        return pk, fmax, fsum

    init = (packed, jnp.full((S, P), NEG, jnp.float32), jnp.zeros((S, P), jnp.float32))
    _, fmax, fsum = jax.lax.fori_loop(0, _K + 1, step, init)
    fmean = fsum * jnp.float32(1.0 / _K)

    Wo = Wo_ref[...]
    acc = jnp.dot(fmax, Wo[:P, :], preferred_element_type=jnp.float32)
    acc = acc + jnp.dot(fmean, Wo[P:2 * P, :], preferred_element_type=jnp.float32)
    acc = acc + jnp.dot(xb, Wo[2 * P:, :], preferred_element_type=jnp.float32)
    out_ref[...] = jnp.tanh(acc + bo_ref[...])


def kernel(x, row_splits, Wp, bp, Ws, bs, Wo, bo):
    N, D = x.shape
    nseg = int(row_splits.shape[0]) - 1
    S = N // nseg
    P = Wp.shape[1]
    ND = Ws.shape[1]
    F = Wo.shape[1]

    grid = (nseg,)
    out = pl.pallas_call(
        _body,
        grid=grid,
        in_specs=[
            pl.BlockSpec((S, D), lambda b: (b, 0)),
            pl.BlockSpec((D, P), lambda b: (0, 0)),
            pl.BlockSpec((1, P), lambda b: (0, 0)),
            pl.BlockSpec((D, ND), lambda b: (0, 0)),
            pl.BlockSpec((1, ND), lambda b: (0, 0)),
            pl.BlockSpec((2 * P + D, F), lambda b: (0, 0)),
            pl.BlockSpec((1, F), lambda b: (0, 0)),
        ],
        out_specs=pl.BlockSpec((S, F), lambda b: (b, 0)),
        out_shape=jax.ShapeDtypeStruct((N, F), jnp.float32),
    )(x, Wp, bp.reshape(1, P), Ws, bs.reshape(1, ND), Wo, bo.reshape(1, F))
    return out


# speedup vs baseline: 21.7274x; 1.1646x over previous
import jax
import jax.numpy as jnp
from jax import lax
from jax.experimental import pallas as pl
from jax.experimental.pallas import tpu as pltpu
from jax.experimental.pallas import tpu_sc as plsc

_K = 40
_NC = 2
_NS = 16
_L = 16
_TW = 32
_CH = 16


def _tc1_body(x_ref, Ws_ref, bs_ref, Wp_ref, bp_ref,
              packed_ref, thresh_ref, feat_ref):
    S = x_ref.shape[0]
    ND = Ws_ref.shape[1]
    xb = x_ref[...]
    coords = jnp.dot(xb, Ws_ref[...],
                     preferred_element_type=jnp.float32) + bs_ref[...]
    feat_ref[...] = jnp.dot(xb, Wp_ref[...],
                            preferred_element_type=jnp.float32) + bp_ref[...]

    d2 = jnp.zeros((S, S), jnp.float32)
    for d in range(ND):
        col = coords[:, d:d + 1]
        diff = col - col.reshape(1, S)
        d2 = d2 + diff * diff

    colids = lax.broadcasted_iota(jnp.int32, (S, S), 1)
    packed = jnp.bitwise_or(
        jnp.bitwise_and(lax.bitcast_convert_type(d2, jnp.int32), ~(S - 1)),
        colids)
    packed_ref[...] = packed

    def bs_step(t, carry):
        lo, hi = carry
        mid = lo + ((hi - lo) >> 1)
        cnt = jnp.sum((packed <= mid).astype(jnp.int32), axis=1, keepdims=True)
        ge = cnt >= _K + 1
        return jnp.where(ge, lo, mid + 1), jnp.where(ge, mid, hi)

    lo0 = jnp.zeros((S, 1), jnp.int32)
    hi0 = jnp.full((S, 1), jnp.int32(0x7FFFFFFF))
    _, hi = lax.fori_loop(0, 31, bs_step, (lo0, hi0))
    rowmin = jnp.min(packed, axis=1, keepdims=True)
    thresh_ref[...] = jnp.concatenate(
        [jnp.broadcast_to(hi, (S, _L)), jnp.broadcast_to(rowmin, (S, _L))],
        axis=1)


def _sc_body(packed_hbm, thresh_hbm, feat_hbm, agg_hbm,
             feat_v, th_v, pk_v, sel_v, out_v):
    NP = feat_hbm.shape[0]
    S = 1024
    P = 32
    N = NP // P
    RW = N // (_NC * _NS)
    NV = S // _L

    wid = lax.axis_index("s") * _NC + lax.axis_index("c")
    base = wid * RW
    seg = base // S
    pltpu.sync_copy(feat_hbm.at[pl.ds(seg * S * P, S * P)], feat_v)
    pltpu.sync_copy(thresh_hbm.at[pl.ds(base * _TW, RW * _TW)], th_v)

    iota = lax.broadcasted_iota(jnp.int32, (_L,), 0)
    HMASK = jnp.int32(~(S - 1))
    NEG = jnp.float32(-3.0e38)

    def chunk_body(g, _):
        row0 = base + g * _CH
        pltpu.sync_copy(packed_hbm.at[pl.ds(row0 * S, _CH * S)], pk_v)
        for r in range(_CH):
            tb = (g * _CH + r) * _TW
            tspl = th_v[pl.ds(tb, _L)]
            minspl = th_v[pl.ds(tb + _L, _L)]

            def selbody(v, off):
                pv = pk_v[pl.ds(r * S + v * _L, _L)]
                m = pv <= tspl
                mi = m.astype(jnp.int32)
                idx = jnp.where(m, off + plsc.cumsum(mi) - 1, 79)
                plsc.store_scatter(sel_v, [idx], pv)
                return jnp.minimum(off + plsc.all_reduce_population_count(m), 64)

            lax.fori_loop(0, NV, selbody, jnp.zeros((_L,), jnp.int32))

            def poolbody(k, carry):
                mx0, mx1, ac0, ac1 = carry
                ps = plsc.load_gather(sel_v, [jnp.full((_L,), k, jnp.int32)])
                j = jnp.bitwise_and(ps, S - 1)
                d2s = plsc.bitcast(jnp.bitwise_and(ps, HMASK), jnp.float32)
                w = jnp.exp(-10.0 * d2s)
                jbase = j * P
                g0 = plsc.load_gather(feat_v, [jbase + iota])
                g1 = plsc.load_gather(feat_v, [jbase + _L + iota])
                wf0 = w * g0
                wf1 = w * g1
                ismin = ps == minspl
                mx0 = jnp.where(ismin, mx0, jnp.maximum(mx0, wf0))
                mx1 = jnp.where(ismin, mx1, jnp.maximum(mx1, wf1))
                ac0 = jnp.where(ismin, ac0, ac0 + wf0)
                ac1 = jnp.where(ismin, ac1, ac1 + wf1)
                return mx0, mx1, ac0, ac1

            negs = jnp.full((_L,), NEG)
            zers = jnp.zeros((_L,), jnp.float32)
            mx0, mx1, ac0, ac1 = lax.fori_loop(
                0, _K + 1, poolbody, (negs, negs, zers, zers))
            ob = r * 4 * _L
            out_v[pl.ds(ob, _L)] = mx0
            out_v[pl.ds(ob + _L, _L)] = mx1
            out_v[pl.ds(ob + 2 * _L, _L)] = ac0 * jnp.float32(1.0 / _K)
            out_v[pl.ds(ob + 3 * _L, _L)] = ac1 * jnp.float32(1.0 / _K)
        pltpu.sync_copy(out_v, agg_hbm.at[pl.ds(row0 * 2 * P, _CH * 2 * P)])
        return 0

    lax.fori_loop(0, RW // _CH, chunk_body, 0)


def _tc2_body(agg_ref, x_ref, Wo_ref, bo_ref, out_ref):
    P2 = agg_ref.shape[1]
    Wo = Wo_ref[...]
    acc = jnp.dot(agg_ref[...], Wo[:P2, :], preferred_element_type=jnp.float32)
    acc = acc + jnp.dot(x_ref[...], Wo[P2:, :],
                        preferred_element_type=jnp.float32)
    out_ref[...] = jnp.tanh(acc + bo_ref[...])


def kernel(x, row_splits, Wp, bp, Ws, bs, Wo, bo):
    N, D = x.shape
    nseg = int(row_splits.shape[0]) - 1
    S = N // nseg
    P = Wp.shape[1]
    ND = Ws.shape[1]
    F = Wo.shape[1]

    packed, thresh, feat = pl.pallas_call(
        _tc1_body,
        grid=(nseg,),
        in_specs=[
            pl.BlockSpec((S, D), lambda b: (b, 0)),
            pl.BlockSpec((D, ND), lambda b: (0, 0)),
            pl.BlockSpec((1, ND), lambda b: (0, 0)),
            pl.BlockSpec((D, P), lambda b: (0, 0)),
            pl.BlockSpec((1, P), lambda b: (0, 0)),
        ],
        out_specs=[
            pl.BlockSpec((S, S), lambda b: (b, 0)),
            pl.BlockSpec((S, _TW), lambda b: (b, 0)),
            pl.BlockSpec((S, P), lambda b: (b, 0)),
        ],
        out_shape=[
            jax.ShapeDtypeStruct((N, S), jnp.int32),
            jax.ShapeDtypeStruct((N, _TW), jnp.int32),
            jax.ShapeDtypeStruct((N, P), jnp.float32),
        ],
    )(x, Ws, bs.reshape(1, ND), Wp, bp.reshape(1, P))

    mesh = plsc.VectorSubcoreMesh(core_axis_name="c", subcore_axis_name="s",
                                  num_cores=_NC, num_subcores=_NS)
    RW = N // (_NC * _NS)
    agg = pl.kernel(
        _sc_body,
        out_type=jax.ShapeDtypeStruct((N * 2 * P,), jnp.float32),
        mesh=mesh,
        compiler_params=pltpu.CompilerParams(needs_layout_passes=False),
        scratch_types=[
            pltpu.VMEM((S * P,), jnp.float32),
            pltpu.VMEM((RW * _TW,), jnp.int32),
            pltpu.VMEM((_CH * S,), jnp.int32),
            pltpu.VMEM((80,), jnp.int32),
            pltpu.VMEM((_CH * 2 * P,), jnp.float32),
        ],
    )(packed.reshape(N * S), thresh.reshape(N * _TW), feat.reshape(N * P))
    agg = agg.reshape(N, 2 * P)

    BR = 2048
    out = pl.pallas_call(
        _tc2_body,
        grid=(N // BR,),
        in_specs=[
            pl.BlockSpec((BR, 2 * P), lambda b: (b, 0)),
            pl.BlockSpec((BR, D), lambda b: (b, 0)),
            pl.BlockSpec((2 * P + D, F), lambda b: (0, 0)),
            pl.BlockSpec((1, F), lambda b: (0, 0)),
        ],
        out_specs=pl.BlockSpec((BR, F), lambda b: (b, 0)),
        out_shape=jax.ShapeDtypeStruct((N, F), jnp.float32),
    )(agg, x, Wo, bo.reshape(1, F))
    return out
